# Initial kernel scaffold; baseline (speedup 1.0000x reference)
import jax
import jax.numpy as jnp
from jax.experimental import pallas as pl


def kernel(nfeat, edge_index, W0, W1, W2, W3, W4, b0, b1, b2, b3, b4, Wp, bp):
    raise NotImplementedError("write your pallas kernel here")


# speedup vs baseline: 5.3732x; 5.3732x over previous
import functools

import jax
import jax.numpy as jnp
from jax import lax
from jax.experimental import pallas as pl
from jax.experimental.pallas import tpu as pltpu
from jax.experimental.pallas import tpu_sc as plsc

N = 10000
NP = 10240
E = 320000
EPT = E // 32
EPTP = 10240
D = 128
L = 5
G = 100
T = 10

C = 128
KB = EPTP // (8 * C)
HB = KB // 2
NPT = NP // 16

_mesh = plsc.VectorSubcoreMesh(core_axis_name="c", subcore_axis_name="s")


@functools.partial(
    pl.kernel,
    out_type=jax.ShapeDtypeStruct((32, NPT, D), jnp.float32),
    mesh=_mesh,
    scratch_types=[
        pltpu.VMEM((HB * 8, C), jnp.int32),
        pltpu.VMEM((C, D), jnp.float32),
        pltpu.VMEM_SHARED((NP, D), jnp.float32),
    ],
)
def _deg_kernel(idxm, deg_out, idx_v, rows_v, agg_sp):
    core = lax.axis_index("c")
    sub = lax.axis_index("s")
    slab = core * 16 + sub
    zero16 = jnp.zeros((16,), jnp.float32)
    one16 = jnp.ones((16,), jnp.float32)

    def zb_body(i, _):
        rows_v[i // 8, pl.ds((i % 8) * 16, 16)] = zero16
        return 0

    lax.fori_loop(0, C * 8, zb_body, 0)

    def zcp_body(k, _):
        pltpu.sync_copy(rows_v, agg_sp.at[pl.ds(sub * NPT + k * C, C)])
        return 0

    lax.fori_loop(0, NPT // C, zcp_body, 0)
    plsc.subcore_barrier()

    def ones_body(i, _):
        rows_v[i // 8, pl.ds((i % 8) * 16, 16)] = one16
        return 0

    lax.fori_loop(0, C * 8, ones_body, 0)

    def outer(h, _):
        base = slab * EPTP + h * (HB * 8 * C)

        def ld_body(r, _):
            pltpu.sync_copy(idxm.at[pl.ds(base + r * C, C)], idx_v.at[r])
            return 0

        lax.fori_loop(0, HB * 8, ld_body, 0)

        def body(i, _):
            pltpu.sync_copy(rows_v, agg_sp.at[idx_v.at[i]], add=True)
            return 0

        lax.fori_loop(0, HB * 8, body, 0)
        return 0

    lax.fori_loop(0, KB // HB, outer, 0)
    plsc.subcore_barrier()

    pltpu.sync_copy(agg_sp.at[pl.ds(sub * NPT, NPT)], deg_out.at[slab])


@functools.partial(
    pl.kernel,
    out_type=jax.ShapeDtypeStruct((32, NPT, D), jnp.float32),
    mesh=_mesh,
    scratch_types=[
        pltpu.VMEM((HB * 8, C), jnp.int32),
        pltpu.VMEM((HB * 8, C), jnp.int32),
        pltpu.VMEM((C, D), jnp.float32),
        pltpu.VMEM_SHARED((NP, D), jnp.float32),
        pltpu.SemaphoreType.DMA,
    ],
)
def _scatter_kernel(u, srcm, dstm, parts_out, src_v, dst_v, rows_v,
                    agg_sp, gsem):
    core = lax.axis_index("c")
    sub = lax.axis_index("s")
    slab = core * 16 + sub
    zero16 = jnp.zeros((16,), jnp.float32)

    def zb_body(i, _):
        rows_v[i // 8, pl.ds((i % 8) * 16, 16)] = zero16
        return 0

    lax.fori_loop(0, C * 8, zb_body, 0)

    def zcp_body(k, _):
        pltpu.sync_copy(rows_v, agg_sp.at[pl.ds(sub * NPT + k * C, C)])
        return 0

    lax.fori_loop(0, NPT // C, zcp_body, 0)
    plsc.subcore_barrier()

    def outer(h, _):
        base = slab * EPTP + h * (HB * 8 * C)

        def ld_body(r, _):
            pltpu.sync_copy(srcm.at[pl.ds(base + r * C, C)], src_v.at[r])
            pltpu.sync_copy(dstm.at[pl.ds(base + r * C, C)], dst_v.at[r])
            return 0

        lax.fori_loop(0, HB * 8, ld_body, 0)

        def body(i, _):
            pltpu.async_copy(u.at[src_v.at[i]], rows_v, gsem).wait()
            pltpu.sync_copy(rows_v, agg_sp.at[dst_v.at[i]], add=True)
            return 0

        lax.fori_loop(0, HB * 8, body, 0)
        return 0

    lax.fori_loop(0, KB // HB, outer, 0)
    plsc.subcore_barrier()

    pltpu.sync_copy(agg_sp.at[pl.ds(sub * NPT, NPT)], parts_out.at[slab])


BNP = 640
GRIDP = NP // BNP
BNL = 400
GRIDL = N // BNL


def _scales(dego_ref, degi_ref):
    do = dego_ref[...]
    di = degi_ref[...]
    deg_o = do[0, :, 0:1] + do[1, :, 0:1]
    deg_i = di[0, :, 0:1] + di[1, :, 0:1]
    dso = lax.rsqrt(jnp.maximum(deg_o, 1.0))
    dsi = lax.rsqrt(jnp.maximum(deg_i, 1.0))
    return dso, dsi


def _tc_first_body(dego_ref, degi_ref, x_ref, w_ref, u_ref):
    dso, _ = _scales(dego_ref, degi_ref)
    u_ref[...] = jnp.dot(x_ref[...] * dso, w_ref[...],
                         preferred_element_type=jnp.float32)


def _tc_mid_body(dego_ref, degi_ref, p_ref, b_ref, w_ref, u_ref):
    dso, dsi = _scales(dego_ref, degi_ref)
    p = p_ref[...]
    h = jax.nn.relu((p[0] + p[1]) * dsi + b_ref[...])
    u_ref[...] = jnp.dot(h * dso, w_ref[...],
                         preferred_element_type=jnp.float32)


def _tc_last_body(dego_ref, degi_ref, p_ref, b_ref, hg_ref):
    i = pl.program_id(0)
    _, dsi = _scales(dego_ref, degi_ref)
    p = p_ref[...]
    z = (p[0] + p[1]) * dsi + b_ref[...]
    blk = jnp.stack([jnp.sum(z[j * 100:(j + 1) * 100, :], axis=0)
                     for j in range(BNL // 100)], axis=0)
    hg_ref[pl.ds(i * (BNL // 100), BNL // 100), :] = blk


def _tc_proj_body(hg_ref, wp_ref, bp_ref, o_ref):
    o_ref[...] = jnp.dot(hg_ref[...], wp_ref[...],
                         preferred_element_type=jnp.float32) + bp_ref[...]


def _deg_spec(bn):
    return pl.BlockSpec((2, bn, D), lambda i: (0, i, 0))


def _mat_spec(bn):
    return pl.BlockSpec((bn, D), lambda i: (i, 0))


def _p_spec(bn):
    return pl.BlockSpec((2, bn, D), lambda i: (0, i, 0))


_w_spec = pl.BlockSpec((D, D), lambda i: (0, 0))
_b_spec = pl.BlockSpec((1, D), lambda i: (0, 0))


def _tc_first(dego, degi, x, w):
    return pl.pallas_call(
        _tc_first_body,
        grid=(GRIDP,),
        in_specs=[_deg_spec(BNP), _deg_spec(BNP), _mat_spec(BNP), _w_spec],
        out_specs=_mat_spec(BNP),
        out_shape=jax.ShapeDtypeStruct((NP, D), jnp.float32),
    )(dego, degi, x, w)


def _tc_mid(dego, degi, p, b, w):
    return pl.pallas_call(
        _tc_mid_body,
        grid=(GRIDP,),
        in_specs=[_deg_spec(BNP), _deg_spec(BNP), _p_spec(BNP), _b_spec,
                  _w_spec],
        out_specs=_mat_spec(BNP),
        out_shape=jax.ShapeDtypeStruct((NP, D), jnp.float32),
    )(dego, degi, p, b, w)


def _tc_last(dego, degi, p, b):
    return pl.pallas_call(
        _tc_last_body,
        grid=(GRIDL,),
        in_specs=[_deg_spec(BNL), _deg_spec(BNL), _p_spec(BNL), _b_spec],
        out_specs=pl.BlockSpec((G, D), lambda i: (0, 0)),
        out_shape=jax.ShapeDtypeStruct((G, D), jnp.float32),
    )(dego, degi, p, b)


def _tc_proj(hg, wp, bp):
    return pl.pallas_call(
        _tc_proj_body,
        in_specs=[pl.BlockSpec((G, D), lambda: (0, 0)),
                  pl.BlockSpec((D, T), lambda: (0, 0)),
                  pl.BlockSpec((1, T), lambda: (0, 0))],
        out_specs=pl.BlockSpec((G, T), lambda: (0, 0)),
        out_shape=jax.ShapeDtypeStruct((G, T), jnp.float32),
    )(hg, wp, bp)


def _pad_edges(idx):
    tiles = idx.reshape(32, EPT)
    pad = jnp.broadcast_to(
        N + jnp.arange(EPTP - EPT, dtype=jnp.int32), (32, EPTP - EPT))
    return jnp.concatenate([tiles, pad], axis=1).reshape(32 * EPTP)


def kernel(nfeat, edge_index, W0, W1, W2, W3, W4, b0, b1, b2, b3, b4, Wp, bp):
    srcm = _pad_edges(edge_index[0])
    dstm = _pad_edges(edge_index[1])
    Ws = [W0, W1, W2, W3, W4]
    bs = [b0.reshape(1, D), b1.reshape(1, D), b2.reshape(1, D),
          b3.reshape(1, D), b4.reshape(1, D)]
    x = jnp.pad(nfeat, ((0, NP - N), (0, 0)))

    dego = _deg_kernel(srcm).reshape(2, NP, D)
    degi = _deg_kernel(dstm).reshape(2, NP, D)

    u = _tc_first(dego, degi, x, Ws[0])
    for layer in range(L - 1):
        parts = _scatter_kernel(u, srcm, dstm).reshape(2, NP, D)
        u = _tc_mid(dego, degi, parts, bs[layer], Ws[layer + 1])
    parts = _scatter_kernel(u, srcm, dstm).reshape(2, NP, D)
    hg = _tc_last(dego, degi, parts, bs[L - 1])
    return _tc_proj(hg, Wp, bp.reshape(1, T))


# speedup vs baseline: 7.6903x; 1.4312x over previous
import functools

import jax
import jax.numpy as jnp
from jax import lax
from jax.experimental import pallas as pl
from jax.experimental.pallas import tpu as pltpu
from jax.experimental.pallas import tpu_sc as plsc

N = 10000
NP = 10240
E = 320000
EPT = E // 32
EPTP = 10240
D = 128
L = 5
G = 100
T = 10

C = 128
KB = EPTP // (8 * C)
SB = 8
NST = EPTP // (SB * C)
NPT = NP // 16

_mesh = plsc.VectorSubcoreMesh(core_axis_name="c", subcore_axis_name="s")


@functools.partial(
    pl.kernel,
    out_type=jax.ShapeDtypeStruct((32, NPT, D), jnp.float32),
    mesh=_mesh,
    scratch_types=[
        pltpu.VMEM((SB * C,), jnp.int32),
        pltpu.VMEM((SB, C), jnp.int32),
        pltpu.VMEM((C, D), jnp.float32),
        pltpu.VMEM_SHARED((NP, D), jnp.float32),
        pltpu.SemaphoreType.DMA,
    ],
)
def _deg_kernel(idxm, deg_out, di1, di2, rows_v, agg_sp, ssem):
    core = lax.axis_index("c")
    sub = lax.axis_index("s")
    slab = core * 16 + sub
    zero16 = jnp.zeros((16,), jnp.float32)
    one16 = jnp.ones((16,), jnp.float32)

    def zb_body(i, _):
        rows_v[i // 8, pl.ds((i % 8) * 16, 16)] = zero16
        return 0

    lax.fori_loop(0, C * 8, zb_body, 0)

    def zcp_body(k, _):
        pltpu.sync_copy(rows_v, agg_sp.at[pl.ds(sub * NPT + k * C, C)])
        return 0

    lax.fori_loop(0, NPT // C, zcp_body, 0)
    plsc.subcore_barrier()

    def ones_body(i, _):
        rows_v[i // 8, pl.ds((i % 8) * 16, 16)] = one16
        return 0

    lax.fori_loop(0, C * 8, ones_body, 0)

    def stage(s, _):
        base = slab * EPTP + s * (SB * C)
        pltpu.sync_copy(idxm.at[pl.ds(base, SB * C)], di1)

        def rp_body(t, _):
            di2[t // 8, pl.ds((t % 8) * 16, 16)] = di1[pl.ds(t * 16, 16)]
            return 0

        lax.fori_loop(0, SB * 8, rp_body, 0)

        descs = [pltpu.async_copy(rows_v, agg_sp.at[di2.at[j]], ssem,
                                  add=True) for j in range(SB)]
        for d in descs:
            d.wait()
        return 0

    lax.fori_loop(0, NST, stage, 0)
    plsc.subcore_barrier()

    pltpu.sync_copy(agg_sp.at[pl.ds(sub * NPT, NPT)], deg_out.at[slab])


@functools.partial(
    pl.kernel,
    out_type=jax.ShapeDtypeStruct((32, NPT, D), jnp.float32),
    mesh=_mesh,
    scratch_types=[
        pltpu.VMEM((SB * C,), jnp.int32),
        pltpu.VMEM((SB * C,), jnp.int32),
        pltpu.VMEM((SB, C), jnp.int32),
        pltpu.VMEM((C, D), jnp.float32),
        pltpu.VMEM((C, D), jnp.float32),
        pltpu.VMEM_SHARED((NP, D), jnp.float32),
        pltpu.SemaphoreType.DMA,
        pltpu.SemaphoreType.DMA,
        pltpu.SemaphoreType.DMA,
    ],
)
def _scatter_kernel(u, srcm, dstm, parts_out, si1, di1, di2, rows0, rows1,
                    agg_sp, sg0, sg1, ssem):
    core = lax.axis_index("c")
    sub = lax.axis_index("s")
    slab = core * 16 + sub
    zero16 = jnp.zeros((16,), jnp.float32)

    def zb_body(i, _):
        rows0[i // 8, pl.ds((i % 8) * 16, 16)] = zero16
        return 0

    lax.fori_loop(0, C * 8, zb_body, 0)

    def zcp_body(k, _):
        pltpu.sync_copy(rows0, agg_sp.at[pl.ds(sub * NPT + k * C, C)])
        return 0

    lax.fori_loop(0, NPT // C, zcp_body, 0)
    plsc.subcore_barrier()

    def stage(s, _):
        base = slab * EPTP + s * (SB * C)
        pltpu.sync_copy(srcm.at[pl.ds(base, SB * C)], si1)
        pltpu.sync_copy(dstm.at[pl.ds(base, SB * C)], di1)

        def rp_body(t, _):
            di2[t // 8, pl.ds((t % 8) * 16, 16)] = di1[pl.ds(t * 16, 16)]
            return 0

        lax.fori_loop(0, SB * 8, rp_body, 0)

        def pair(j, _):
            i0 = 2 * j
            i1 = 2 * j + 1
            g0 = pltpu.async_copy(u.at[si1.at[pl.ds(i0 * C, C)]], rows0, sg0)
            g1 = pltpu.async_copy(u.at[si1.at[pl.ds(i1 * C, C)]], rows1, sg1)
            g0.wait()
            s0 = pltpu.async_copy(rows0, agg_sp.at[di2.at[i0]], ssem,
                                  add=True)
            g1.wait()
            s1 = pltpu.async_copy(rows1, agg_sp.at[di2.at[i1]], ssem,
                                  add=True)
            s0.wait()
            s1.wait()
            return 0

        lax.fori_loop(0, SB // 2, pair, 0)
        return 0

    lax.fori_loop(0, NST, stage, 0)
    plsc.subcore_barrier()

    pltpu.sync_copy(agg_sp.at[pl.ds(sub * NPT, NPT)], parts_out.at[slab])


BNP = 640
GRIDP = NP // BNP
BNL = 400
GRIDL = N // BNL


def _scales(dego_ref, degi_ref):
    do = dego_ref[...]
    di = degi_ref[...]
    deg_o = do[0, :, 0:1] + do[1, :, 0:1]
    deg_i = di[0, :, 0:1] + di[1, :, 0:1]
    dso = lax.rsqrt(jnp.maximum(deg_o, 1.0))
    dsi = lax.rsqrt(jnp.maximum(deg_i, 1.0))
    return dso, dsi


def _tc_first_body(dego_ref, degi_ref, x_ref, w_ref, u_ref):
    dso, _ = _scales(dego_ref, degi_ref)
    u_ref[...] = jnp.dot(x_ref[...] * dso, w_ref[...],
                         preferred_element_type=jnp.float32)


def _tc_mid_body(dego_ref, degi_ref, p_ref, b_ref, w_ref, u_ref):
    dso, dsi = _scales(dego_ref, degi_ref)
    p = p_ref[...]
    h = jax.nn.relu((p[0] + p[1]) * dsi + b_ref[...])
    u_ref[...] = jnp.dot(h * dso, w_ref[...],
                         preferred_element_type=jnp.float32)


def _tc_last_body(dego_ref, degi_ref, p_ref, b_ref, hg_ref):
    i = pl.program_id(0)
    _, dsi = _scales(dego_ref, degi_ref)
    p = p_ref[...]
    z = (p[0] + p[1]) * dsi + b_ref[...]
    blk = jnp.stack([jnp.sum(z[j * 100:(j + 1) * 100, :], axis=0)
                     for j in range(BNL // 100)], axis=0)
    hg_ref[pl.ds(i * (BNL // 100), BNL // 100), :] = blk


def _tc_proj_body(hg_ref, wp_ref, bp_ref, o_ref):
    o_ref[...] = jnp.dot(hg_ref[...], wp_ref[...],
                         preferred_element_type=jnp.float32) + bp_ref[...]


def _deg_spec(bn):
    return pl.BlockSpec((2, bn, D), lambda i: (0, i, 0))


def _mat_spec(bn):
    return pl.BlockSpec((bn, D), lambda i: (i, 0))


def _p_spec(bn):
    return pl.BlockSpec((2, bn, D), lambda i: (0, i, 0))


_w_spec = pl.BlockSpec((D, D), lambda i: (0, 0))
_b_spec = pl.BlockSpec((1, D), lambda i: (0, 0))


def _tc_first(dego, degi, x, w):
    return pl.pallas_call(
        _tc_first_body,
        grid=(GRIDP,),
        in_specs=[_deg_spec(BNP), _deg_spec(BNP), _mat_spec(BNP), _w_spec],
        out_specs=_mat_spec(BNP),
        out_shape=jax.ShapeDtypeStruct((NP, D), jnp.float32),
    )(dego, degi, x, w)


def _tc_mid(dego, degi, p, b, w):
    return pl.pallas_call(
        _tc_mid_body,
        grid=(GRIDP,),
        in_specs=[_deg_spec(BNP), _deg_spec(BNP), _p_spec(BNP), _b_spec,
                  _w_spec],
        out_specs=_mat_spec(BNP),
        out_shape=jax.ShapeDtypeStruct((NP, D), jnp.float32),
    )(dego, degi, p, b, w)


def _tc_last(dego, degi, p, b):
    return pl.pallas_call(
        _tc_last_body,
        grid=(GRIDL,),
        in_specs=[_deg_spec(BNL), _deg_spec(BNL), _p_spec(BNL), _b_spec],
        out_specs=pl.BlockSpec((G, D), lambda i: (0, 0)),
        out_shape=jax.ShapeDtypeStruct((G, D), jnp.float32),
    )(dego, degi, p, b)


def _tc_proj(hg, wp, bp):
    return pl.pallas_call(
        _tc_proj_body,
        in_specs=[pl.BlockSpec((G, D), lambda: (0, 0)),
                  pl.BlockSpec((D, T), lambda: (0, 0)),
                  pl.BlockSpec((1, T), lambda: (0, 0))],
        out_specs=pl.BlockSpec((G, T), lambda: (0, 0)),
        out_shape=jax.ShapeDtypeStruct((G, T), jnp.float32),
    )(hg, wp, bp)


def _pad_edges(idx):
    tiles = idx.reshape(32, EPT)
    pad = jnp.broadcast_to(
        N + jnp.arange(EPTP - EPT, dtype=jnp.int32), (32, EPTP - EPT))
    return jnp.concatenate([tiles, pad], axis=1).reshape(32 * EPTP)


def kernel(nfeat, edge_index, W0, W1, W2, W3, W4, b0, b1, b2, b3, b4, Wp, bp):
    srcm = _pad_edges(edge_index[0])
    dstm = _pad_edges(edge_index[1])
    Ws = [W0, W1, W2, W3, W4]
    bs = [b0.reshape(1, D), b1.reshape(1, D), b2.reshape(1, D),
          b3.reshape(1, D), b4.reshape(1, D)]
    x = jnp.pad(nfeat, ((0, NP - N), (0, 0)))

    dego = _deg_kernel(srcm).reshape(2, NP, D)
    degi = _deg_kernel(dstm).reshape(2, NP, D)

    u = _tc_first(dego, degi, x, Ws[0])
    for layer in range(L - 1):
        parts = _scatter_kernel(u, srcm, dstm).reshape(2, NP, D)
        u = _tc_mid(dego, degi, parts, bs[layer], Ws[layer + 1])
    parts = _scatter_kernel(u, srcm, dstm).reshape(2, NP, D)
    hg = _tc_last(dego, degi, parts, bs[L - 1])
    return _tc_proj(hg, Wp, bp.reshape(1, T))
